# baseline (speedup 1.0000x reference)
import functools

import jax
import jax.numpy as jnp
from jax import lax
from jax.experimental import pallas as pl
from jax.experimental.pallas import tpu as pltpu
from jax.experimental.pallas import tpu_sc as plsc

NC = 2
NS = 16
NW = NC * NS
CHUNK = 64
NBUF = 4


def _sc_degree(dst_blocks, ones_c, zeros_flat, n_acc, k_per_core):
    K = dst_blocks.shape[1]
    rpt = n_acc // NS

    @functools.partial(
        pl.kernel,
        out_type=jax.ShapeDtypeStruct((NC * n_acc,), jnp.float32),
        mesh=plsc.VectorSubcoreMesh(core_axis_name="c", subcore_axis_name="s"),
        scratch_types=[
            pltpu.VMEM((K, CHUNK), jnp.int32),
            pltpu.VMEM((CHUNK,), jnp.float32),
            pltpu.VMEM((rpt,), jnp.float32),
            pltpu.VMEM_SHARED((n_acc,), jnp.float32),
        ],
    )
    def deg_kernel(dst_hbm, ones_hbm, zeros_hbm, out_hbm, dst_v, ones_v,
                   stage_v, deg_sh):
        c = lax.axis_index("c")
        s = lax.axis_index("s")
        wid = c * NS + s
        kc = jnp.where(c == 0, k_per_core[0], k_per_core[1])
        pltpu.sync_copy(zeros_hbm, stage_v)
        pltpu.sync_copy(stage_v, deg_sh.at[pl.ds(s * rpt, rpt)])
        pltpu.sync_copy(dst_hbm.at[wid], dst_v)
        pltpu.sync_copy(ones_hbm, ones_v)
        plsc.subcore_barrier()

        def body(j, carry):
            pltpu.sync_copy(ones_v, deg_sh.at[dst_v.at[j]], add=True)
            return carry

        lax.fori_loop(0, kc, body, 0)
        plsc.subcore_barrier()
        pltpu.sync_copy(deg_sh.at[pl.ds(s * rpt, rpt)], stage_v)
        pltpu.sync_copy(stage_v, out_hbm.at[pl.ds(c * n_acc + s * rpt, rpt)])

    return deg_kernel(dst_blocks, ones_c, zeros_flat)


def _sc_aggregate(hs, src_blocks, dst_blocks, zeros_rows, n_acc, k_per_core):
    H = hs.shape[1]
    rpt = n_acc // NS

    assert min(k_per_core) >= 2 and rpt % CHUNK == 0
    assert all(k % NBUF == 0 for k in k_per_core)

    @functools.partial(
        pl.kernel,
        out_type=jax.ShapeDtypeStruct((NC * n_acc, H), jnp.float32),
        mesh=plsc.VectorSubcoreMesh(core_axis_name="c", subcore_axis_name="s"),
        scratch_types=[
            pltpu.VMEM((NBUF, CHUNK), jnp.int32),
            pltpu.VMEM((NBUF, CHUNK), jnp.int32),
            pltpu.VMEM((NBUF, CHUNK, H), jnp.float32),
            pltpu.VMEM_SHARED((n_acc, H), jnp.float32),
        ] + [pltpu.SemaphoreType.DMA] * (3 * NBUF),
    )
    def agg_kernel(hs_hbm, src_hbm, dst_hbm, zrows_hbm, out_hbm,
                   src_v, dst_v, rows_v, acc_sh, *sems):
        isem = sems[:NBUF]
        dsem = sems[NBUF:2 * NBUF]
        gsem = sems[2 * NBUF:]
        c = lax.axis_index("c")
        s = lax.axis_index("s")
        wid = c * NS + s
        kc = jnp.where(c == 0, k_per_core[0], k_per_core[1])
        pltpu.sync_copy(zrows_hbm, rows_v.at[0])
        for z in range(rpt // CHUNK):
            pltpu.sync_copy(rows_v.at[0],
                            acc_sh.at[pl.ds(s * rpt + z * CHUNK, CHUNK)])
        plsc.subcore_barrier()

        def src_start(j, bi):
            pltpu.async_copy(src_hbm.at[wid, j], src_v.at[bi], isem[bi])

        def src_wait(j, bi):
            pltpu.make_async_copy(src_hbm.at[wid, j], src_v.at[bi],
                                  isem[bi]).wait()

        def dst_start(j, bi):
            pltpu.async_copy(dst_hbm.at[wid, j], dst_v.at[bi], dsem[bi])

        def dst_wait(j, bi):
            pltpu.make_async_copy(dst_hbm.at[wid, j], dst_v.at[bi],
                                  dsem[bi]).wait()

        def gather_start(bi):
            pltpu.async_copy(hs_hbm.at[src_v.at[bi]], rows_v.at[bi],
                             gsem[bi])

        def gather_wait(bi):
            pltpu.make_async_copy(hs_hbm.at[src_v.at[bi]], rows_v.at[bi],
                                  gsem[bi]).wait()

        for bi in range(NBUF):
            src_start(bi, bi)
            dst_start(bi, bi)
        for bi in range(NBUF - 1):
            src_wait(bi, bi)
            gather_start(bi)

        def group(g, carry):
            for bi in range(NBUF):
                j = g * NBUF + bi
                bn = (bi + NBUF - 1) % NBUF

                @pl.when(j + NBUF - 1 < kc)
                def _():
                    src_wait(j + NBUF - 1, bn)
                    gather_start(bn)

                gather_wait(bi)
                dst_wait(j, bi)
                pltpu.sync_copy(rows_v.at[bi], acc_sh.at[dst_v.at[bi]],
                                add=True)

                @pl.when(j + NBUF < kc)
                def _():
                    src_start(j + NBUF, bi)
                    dst_start(j + NBUF, bi)
            return carry

        lax.fori_loop(0, kc // NBUF, group, 0)
        plsc.subcore_barrier()
        for z in range(rpt // CHUNK):
            bi = z % NBUF
            pltpu.sync_copy(acc_sh.at[pl.ds(s * rpt + z * CHUNK, CHUNK)],
                            rows_v.at[bi])
            pltpu.sync_copy(
                rows_v.at[bi],
                out_hbm.at[pl.ds(c * n_acc + s * rpt + z * CHUNK, CHUNK)])

    return agg_kernel(hs, src_blocks, dst_blocks, zeros_rows)


def _tc_prep(x, W, deg0, deg1, rows_blk):
    N, D = x.shape
    H = W.shape[1]
    grid = N // rows_blk

    def body(x_ref, w_ref, d0_ref, d1_ref, hs_ref, dinv_ref):
        dinv = lax.rsqrt(d0_ref[...] + d1_ref[...] + 1.0)
        h = jnp.dot(x_ref[...], w_ref[...],
                    preferred_element_type=jnp.float32)
        hs_ref[...] = h * dinv
        dinv_ref[...] = dinv

    return pl.pallas_call(
        body,
        grid=(grid,),
        in_specs=[
            pl.BlockSpec((rows_blk, D), lambda i: (i, 0)),
            pl.BlockSpec((D, H), lambda i: (0, 0)),
            pl.BlockSpec((rows_blk, 1), lambda i: (i, 0)),
            pl.BlockSpec((rows_blk, 1), lambda i: (i, 0)),
        ],
        out_specs=[
            pl.BlockSpec((rows_blk, H), lambda i: (i, 0)),
            pl.BlockSpec((rows_blk, 1), lambda i: (i, 0)),
        ],
        out_shape=[
            jax.ShapeDtypeStruct((N, H), jnp.float32),
            jax.ShapeDtypeStruct((N, 1), jnp.float32),
        ],
    )(x, W, deg0, deg1)


def _tc_epilogue(acc0, acc1, hs, dinv, b2, rows_blk):
    N, H = hs.shape
    grid = N // rows_blk

    def body(a0_ref, a1_ref, hs_ref, dinv_ref, b_ref, out_ref):
        t = (a0_ref[...] + a1_ref[...] + hs_ref[...]) * dinv_ref[...]
        t = t + b_ref[...]
        out_ref[...] = jnp.where(t > 0.0, t,
                                 jnp.exp(jnp.minimum(t, 0.0)) - 1.0)

    return pl.pallas_call(
        body,
        grid=(grid,),
        in_specs=[
            pl.BlockSpec((rows_blk, H), lambda i: (i, 0)),
            pl.BlockSpec((rows_blk, H), lambda i: (i, 0)),
            pl.BlockSpec((rows_blk, H), lambda i: (i, 0)),
            pl.BlockSpec((rows_blk, 1), lambda i: (i, 0)),
            pl.BlockSpec((1, H), lambda i: (0, 0)),
        ],
        out_specs=pl.BlockSpec((rows_blk, H), lambda i: (i, 0)),
        out_shape=jax.ShapeDtypeStruct((N, H), jnp.float32),
    )(acc0, acc1, hs, dinv, b2)


def kernel(x, edge_index, W, b):
    N, D = x.shape
    H = W.shape[1]
    E = edge_index.shape[1]

    frac1 = 0.5
    e1_pt = ((int(E * frac1) // NS) // (CHUNK * NBUF)) * (CHUNK * NBUF)
    e1_pt = max(e1_pt, CHUNK * NBUF)
    K1 = e1_pt // CHUNK
    e0_pt_real = (E - NS * e1_pt) // NS
    K0 = ((e0_pt_real + CHUNK * NBUF - 1) // (CHUNK * NBUF)) * NBUF
    n_acc = ((N + 1 + NS * CHUNK - 1) // (NS * CHUNK)) * (NS * CHUNK)

    src = edge_index[0]
    dst = edge_index[1]

    def blockify(seg, n_pt, k_chunks, fill):
        a = seg.reshape(NS, n_pt)
        a = jnp.pad(a, ((0, 0), (0, k_chunks * CHUNK - n_pt)),
                    constant_values=fill)
        return a.reshape(NS, k_chunks, CHUNK)

    e0_tot = NS * e0_pt_real
    blocks = []
    for arr, fill in ((src, 0), (dst, N)):
        b0 = blockify(arr[:e0_tot], e0_pt_real, K0, fill)
        b1 = blockify(arr[e0_tot:], e1_pt, K1, fill)
        b1 = jnp.pad(b1, ((0, 0), (0, K0 - K1), (0, 0)),
                     constant_values=fill)
        blocks.append(jnp.concatenate([b0, b1], axis=0))
    src_blocks, dst_blocks = blocks
    k_per_core = (K0, K1)

    ones_c = jnp.ones((CHUNK,), jnp.float32)
    zeros_flat = jnp.zeros((n_acc // NS,), jnp.float32)
    zeros_rows = jnp.zeros((CHUNK, H), jnp.float32)

    deg_flat = _sc_degree(dst_blocks, ones_c, zeros_flat, n_acc, k_per_core)
    deg0 = deg_flat[:N].reshape(N, 1)
    deg1 = deg_flat[n_acc:n_acc + N].reshape(N, 1)

    hs, dinv = _tc_prep(x, W, deg0, deg1, rows_blk=2000)

    acc = _sc_aggregate(hs, src_blocks, dst_blocks, zeros_rows, n_acc,
                        k_per_core)
    acc0 = acc[:N]
    acc1 = acc[n_acc:n_acc + N]

    b2 = b.reshape(1, H)
    return _tc_epilogue(acc0, acc1, hs, dinv, b2, rows_blk=2000)


# speedup vs baseline: 1.1225x; 1.1225x over previous
import functools

import jax
import jax.numpy as jnp
from jax import lax
from jax.experimental import pallas as pl
from jax.experimental.pallas import tpu as pltpu
from jax.experimental.pallas import tpu_sc as plsc

NC = 2
NS = 16
NW = NC * NS
CHUNK = 128
NBUF = 2


def _sc_degree(dst_blocks, ones_c, zeros_flat, n_acc, k_per_core):
    K = dst_blocks.shape[1]
    rpt = n_acc // NS

    @functools.partial(
        pl.kernel,
        out_type=jax.ShapeDtypeStruct((NC * n_acc,), jnp.float32),
        mesh=plsc.VectorSubcoreMesh(core_axis_name="c", subcore_axis_name="s"),
        scratch_types=[
            pltpu.VMEM((K, CHUNK), jnp.int32),
            pltpu.VMEM((CHUNK,), jnp.float32),
            pltpu.VMEM((rpt,), jnp.float32),
            pltpu.VMEM_SHARED((n_acc,), jnp.float32),
        ],
    )
    def deg_kernel(dst_hbm, ones_hbm, zeros_hbm, out_hbm, dst_v, ones_v,
                   stage_v, deg_sh):
        c = lax.axis_index("c")
        s = lax.axis_index("s")
        wid = c * NS + s
        kc = jnp.where(c == 0, k_per_core[0], k_per_core[1])
        pltpu.sync_copy(zeros_hbm, stage_v)
        pltpu.sync_copy(stage_v, deg_sh.at[pl.ds(s * rpt, rpt)])
        pltpu.sync_copy(dst_hbm.at[wid], dst_v)
        pltpu.sync_copy(ones_hbm, ones_v)
        plsc.subcore_barrier()

        def body(j, carry):
            pltpu.sync_copy(ones_v, deg_sh.at[dst_v.at[j]], add=True)
            return carry

        lax.fori_loop(0, kc, body, 0)
        plsc.subcore_barrier()
        pltpu.sync_copy(deg_sh.at[pl.ds(s * rpt, rpt)], stage_v)
        pltpu.sync_copy(stage_v, out_hbm.at[pl.ds(c * n_acc + s * rpt, rpt)])

    return deg_kernel(dst_blocks, ones_c, zeros_flat)


def _sc_aggregate(hs, src_blocks, dst_blocks, zeros_rows, n_acc, k_per_core):
    H = hs.shape[1]
    rpt = n_acc // NS

    assert min(k_per_core) >= 2 and rpt % CHUNK == 0
    assert all(k % NBUF == 0 for k in k_per_core)

    @functools.partial(
        pl.kernel,
        out_type=jax.ShapeDtypeStruct((NC * n_acc, H), jnp.float32),
        mesh=plsc.VectorSubcoreMesh(core_axis_name="c", subcore_axis_name="s"),
        scratch_types=[
            pltpu.VMEM((NBUF, CHUNK), jnp.int32),
            pltpu.VMEM((NBUF, CHUNK), jnp.int32),
            pltpu.VMEM((NBUF, CHUNK, H), jnp.float32),
            pltpu.VMEM_SHARED((n_acc, H), jnp.float32),
        ] + [pltpu.SemaphoreType.DMA] * (3 * NBUF),
    )
    def agg_kernel(hs_hbm, src_hbm, dst_hbm, zrows_hbm, out_hbm,
                   src_v, dst_v, rows_v, acc_sh, *sems):
        isem = sems[:NBUF]
        dsem = sems[NBUF:2 * NBUF]
        gsem = sems[2 * NBUF:]
        c = lax.axis_index("c")
        s = lax.axis_index("s")
        wid = c * NS + s
        kc = jnp.where(c == 0, k_per_core[0], k_per_core[1])
        pltpu.sync_copy(zrows_hbm, rows_v.at[0])
        for z in range(rpt // CHUNK):
            pltpu.sync_copy(rows_v.at[0],
                            acc_sh.at[pl.ds(s * rpt + z * CHUNK, CHUNK)])
        plsc.subcore_barrier()

        def src_start(j, bi):
            pltpu.async_copy(src_hbm.at[wid, j], src_v.at[bi], isem[bi])

        def src_wait(j, bi):
            pltpu.make_async_copy(src_hbm.at[wid, j], src_v.at[bi],
                                  isem[bi]).wait()

        def dst_start(j, bi):
            pltpu.async_copy(dst_hbm.at[wid, j], dst_v.at[bi], dsem[bi])

        def dst_wait(j, bi):
            pltpu.make_async_copy(dst_hbm.at[wid, j], dst_v.at[bi],
                                  dsem[bi]).wait()

        def gather_start(bi):
            pltpu.async_copy(hs_hbm.at[src_v.at[bi]], rows_v.at[bi],
                             gsem[bi])

        def gather_wait(bi):
            pltpu.make_async_copy(hs_hbm.at[src_v.at[bi]], rows_v.at[bi],
                                  gsem[bi]).wait()

        for bi in range(NBUF):
            src_start(bi, bi)
            dst_start(bi, bi)
        for bi in range(NBUF - 1):
            src_wait(bi, bi)
            gather_start(bi)

        def group(g, carry):
            for bi in range(NBUF):
                j = g * NBUF + bi
                bn = (bi + NBUF - 1) % NBUF

                @pl.when(j + NBUF - 1 < kc)
                def _():
                    src_wait(j + NBUF - 1, bn)
                    gather_start(bn)

                gather_wait(bi)
                dst_wait(j, bi)
                pltpu.sync_copy(rows_v.at[bi], acc_sh.at[dst_v.at[bi]],
                                add=True)

                @pl.when(j + NBUF < kc)
                def _():
                    src_start(j + NBUF, bi)
                    dst_start(j + NBUF, bi)
            return carry

        lax.fori_loop(0, kc // NBUF, group, 0)
        plsc.subcore_barrier()
        for z in range(rpt // CHUNK):
            bi = z % NBUF
            pltpu.sync_copy(acc_sh.at[pl.ds(s * rpt + z * CHUNK, CHUNK)],
                            rows_v.at[bi])
            pltpu.sync_copy(
                rows_v.at[bi],
                out_hbm.at[pl.ds(c * n_acc + s * rpt + z * CHUNK, CHUNK)])

    return agg_kernel(hs, src_blocks, dst_blocks, zeros_rows)


def _tc_prep(x, W, deg0, deg1, rows_blk):
    N, D = x.shape
    H = W.shape[1]
    grid = N // rows_blk

    def body(x_ref, w_ref, d0_ref, d1_ref, hs_ref, dinv_ref):
        dinv = lax.rsqrt(d0_ref[...] + d1_ref[...] + 1.0)
        h = jnp.dot(x_ref[...], w_ref[...],
                    preferred_element_type=jnp.float32)
        hs_ref[...] = h * dinv
        dinv_ref[...] = dinv

    return pl.pallas_call(
        body,
        grid=(grid,),
        in_specs=[
            pl.BlockSpec((rows_blk, D), lambda i: (i, 0)),
            pl.BlockSpec((D, H), lambda i: (0, 0)),
            pl.BlockSpec((rows_blk, 1), lambda i: (i, 0)),
            pl.BlockSpec((rows_blk, 1), lambda i: (i, 0)),
        ],
        out_specs=[
            pl.BlockSpec((rows_blk, H), lambda i: (i, 0)),
            pl.BlockSpec((rows_blk, 1), lambda i: (i, 0)),
        ],
        out_shape=[
            jax.ShapeDtypeStruct((N, H), jnp.float32),
            jax.ShapeDtypeStruct((N, 1), jnp.float32),
        ],
    )(x, W, deg0, deg1)


def _tc_epilogue(acc, hs, dinv, b2, n_acc, rows_blk):
    NP, H = hs.shape
    grid = NP // rows_blk
    off = n_acc // rows_blk

    def body(a0_ref, a1_ref, hs_ref, dinv_ref, b_ref, out_ref):
        t = (a0_ref[...] + a1_ref[...] + hs_ref[...]) * dinv_ref[...]
        t = t + b_ref[...]
        out_ref[...] = jnp.where(t > 0.0, t,
                                 jnp.exp(jnp.minimum(t, 0.0)) - 1.0)

    return pl.pallas_call(
        body,
        grid=(grid,),
        in_specs=[
            pl.BlockSpec((rows_blk, H), lambda i: (i, 0)),
            pl.BlockSpec((rows_blk, H), lambda i: (off + i, 0)),
            pl.BlockSpec((rows_blk, H), lambda i: (i, 0)),
            pl.BlockSpec((rows_blk, 1), lambda i: (i, 0)),
            pl.BlockSpec((1, H), lambda i: (0, 0)),
        ],
        out_specs=pl.BlockSpec((rows_blk, H), lambda i: (i, 0)),
        out_shape=jax.ShapeDtypeStruct((NP, H), jnp.float32),
    )(acc, acc, hs, dinv, b2)


def kernel(x, edge_index, W, b):
    N, D = x.shape
    H = W.shape[1]
    E = edge_index.shape[1]

    frac1 = 0.5
    e1_pt = ((int(E * frac1) // NS) // (CHUNK * NBUF)) * (CHUNK * NBUF)
    e1_pt = max(e1_pt, CHUNK * NBUF)
    K1 = e1_pt // CHUNK
    e0_pt_real = (E - NS * e1_pt) // NS
    K0 = ((e0_pt_real + CHUNK * NBUF - 1) // (CHUNK * NBUF)) * NBUF
    n_acc = ((N + 1 + NS * CHUNK - 1) // (NS * CHUNK)) * (NS * CHUNK)

    src = edge_index[0]
    dst = edge_index[1]

    def blockify(seg, n_pt, k_chunks, fill):
        a = seg.reshape(NS, n_pt)
        a = jnp.pad(a, ((0, 0), (0, k_chunks * CHUNK - n_pt)),
                    constant_values=fill)
        return a.reshape(NS, k_chunks, CHUNK)

    e0_tot = NS * e0_pt_real
    blocks = []
    for arr, fill in ((src, 0), (dst, N)):
        b0 = blockify(arr[:e0_tot], e0_pt_real, K0, fill)
        b1 = blockify(arr[e0_tot:], e1_pt, K1, fill)
        b1 = jnp.pad(b1, ((0, 0), (0, K0 - K1), (0, 0)),
                     constant_values=fill)
        blocks.append(jnp.concatenate([b0, b1], axis=0))
    src_blocks, dst_blocks = blocks
    k_per_core = (K0, K1)

    ones_c = jnp.ones((CHUNK,), jnp.float32)
    zeros_flat = jnp.zeros((n_acc // NS,), jnp.float32)
    zeros_rows = jnp.zeros((CHUNK, H), jnp.float32)

    deg_flat = _sc_degree(dst_blocks, ones_c, zeros_flat, n_acc, k_per_core)
    deg0 = deg_flat[:n_acc].reshape(n_acc, 1)
    deg1 = deg_flat[n_acc:].reshape(n_acc, 1)

    x_pad = jnp.pad(x, ((0, n_acc - N), (0, 0)))
    hs, dinv = _tc_prep(x_pad, W, deg0, deg1, rows_blk=1024)

    acc = _sc_aggregate(hs, src_blocks, dst_blocks, zeros_rows, n_acc,
                        k_per_core)

    b2 = b.reshape(1, H)
    out = _tc_epilogue(acc, hs, dinv, b2, n_acc, rows_blk=1024)
    return out[:N]
